# baseline (speedup 1.0000x reference)
import functools
import jax
import jax.numpy as jnp
from jax import lax
from jax.experimental import pallas as pl
from jax.experimental.pallas import tpu as pltpu
from jax.experimental.pallas import tpu_sc as plsc

_IC = 128
_FK = 16


def _build(B, V, D):
    info = plsc.get_sparse_core_info()
    NC, NS, L = info.num_cores, info.num_subcores, info.num_lanes
    NW = NC * NS
    b_per_w = B // NW
    words = b_per_w * D

    mesh = plsc.VectorSubcoreMesh(core_axis_name="c", subcore_axis_name="s")

    @functools.partial(
        pl.kernel,
        mesh=mesh,
        out_type=jax.ShapeDtypeStruct((D * B,), jnp.float32),
        scratch_types=[
            pltpu.VMEM((b_per_w,), jnp.int32),
            pltpu.VMEM((words,), jnp.int32),
            pltpu.VMEM((words,), jnp.float32),
            pltpu.SemaphoreType.DMA,
        ],
        compiler_params=pltpu.CompilerParams(use_tc_tiling_on_sc=True),
    )
    def k(idx_hbm, flat_hbm, out_hbm, idx_v, off_v, stag_v, sem):
        wid = lax.axis_index("s") * NC + lax.axis_index("c")
        base = wid * b_per_w
        pltpu.sync_copy(idx_hbm.at[pl.ds(base, b_per_w)], idx_v)

        def mkoff(d, _):
            def inner(j, _):
                off_v[pl.ds(d * b_per_w + j * L, L)] = (
                    idx_v[pl.ds(j * L, L)] + d * V
                )
                return _

            return lax.fori_loop(0, b_per_w // L, inner, _, unroll=4)

        lax.fori_loop(0, D, mkoff, None)

        n_chunks = words // _IC

        def group(g, _):
            copies = []
            for j in range(_FK):
                u = g * _FK + j
                copies.append(
                    pltpu.async_copy(
                        flat_hbm.at[off_v.at[pl.ds(u * _IC, _IC)]],
                        stag_v.at[pl.ds(u * _IC, _IC)],
                        sem,
                    )
                )
            for cp in copies:
                cp.wait()
            return _

        lax.fori_loop(0, n_chunks // _FK, group, None)

        def wr(d, _):
            pltpu.sync_copy(
                stag_v.at[pl.ds(d * b_per_w, b_per_w)],
                out_hbm.at[pl.ds(d * B + base, b_per_w)],
            )
            return _

        lax.fori_loop(0, D, wr, None)

    return k


def kernel(global_state, embed_weight):
    B, = global_state.shape
    V, D = embed_weight.shape
    flat = embed_weight.T.reshape(-1)
    out1 = _build(B, V, D)(global_state.astype(jnp.int32), flat)
    return out1.reshape(D, B).T


# speedup vs baseline: 11.0616x; 11.0616x over previous
import functools
import jax
import jax.numpy as jnp
from jax import lax
from jax.experimental import pallas as pl
from jax.experimental.pallas import tpu as pltpu
from jax.experimental.pallas import tpu_sc as plsc

_BLK = 128
_STRIPE_BLKS = 244
_RING = 128


def _build(B, V, D):
    info = plsc.get_sparse_core_info()
    NC, NS, L = info.num_cores, info.num_subcores, info.num_lanes
    NW = NC * NS
    stripe_v = _STRIPE_BLKS * _BLK
    tail0 = (V // _BLK) * _BLK
    cap = B + _RING

    mesh = plsc.VectorSubcoreMesh(core_axis_name="c", subcore_axis_name="s")

    @functools.partial(
        pl.kernel,
        mesh=mesh,
        out_type=jax.ShapeDtypeStruct((B * 128,), jnp.float32),
        scratch_types=[
            pltpu.VMEM((B,), jnp.int32),
            pltpu.VMEM((cap,), jnp.int32),
            pltpu.VMEM((cap,), jnp.int32),
            pltpu.VMEM((cap,), jnp.int32),
            pltpu.VMEM((cap,), jnp.int32),
            pltpu.VMEM((64, _BLK), jnp.float32),
            pltpu.VMEM((V - tail0, 64), jnp.float32),
            pltpu.VMEM((_RING * 64,), jnp.float32),
            pltpu.SMEM((64,), jnp.int32),
            pltpu.SMEM((64,), jnp.int32),
            pltpu.SemaphoreType.DMA,
            pltpu.SemaphoreType.DMA,
        ],
        compiler_params=pltpu.CompilerParams(needs_layout_passes=False),
    )
    def k(idx_hbm, wt_hbm, tail_hbm, out_hbm,
          idx_v, pv, pb, wv, wb, chunk, tailb, stag, sv, sb, lsem, wsem):
        wid = lax.axis_index("s") * NC + lax.axis_index("c")
        pltpu.sync_copy(idx_hbm, idx_v)
        pltpu.sync_copy(tail_hbm, tailb)

        iota = lax.iota(jnp.int32, L)

        def prescan(j, cnt):
            v = idx_v[pl.ds(pl.multiple_of(j * L, L), L)]
            q = jnp.minimum(v // stripe_v, NW - 1)
            m = q == wid
            plsc.store_compressed(pv.at[pl.ds(cnt, L)], v, mask=m)
            plsc.store_compressed(pb.at[pl.ds(cnt, L)], j * L + iota, mask=m)
            return cnt + jnp.sum(1 - jnp.minimum(jnp.abs(q - wid), 1))

        n_w = lax.fori_loop(0, B // L, prescan, 0)
        n_scan = pl.cdiv(n_w, L)

        def pair_batch(m_batch, v0, fired, from_tail):
            def one(i, fired):
                al = pl.multiple_of((i >> 4) * L, L)
                lane = jnp.full((L,), i & (L - 1), jnp.int32)
                v_spl = jnp.take(wv[pl.ds(al, L)], lane)
                b_s = jnp.take(wb[pl.ds(al, L)], lane)[0]
                slot = lax.rem(fired, _RING)

                @pl.when(fired >= _RING)
                def _():
                    pltpu.make_async_copy(
                        out_hbm.at[pl.ds(0, 64)],
                        stag.at[pl.ds(0, 64)],
                        wsem,
                    ).wait()

                for j in range(4):
                    if from_tail:
                        row = plsc.load_gather(
                            tailb, [v_spl - v0, iota + j * L])
                    else:
                        row = plsc.load_gather(
                            chunk, [iota + j * L, v_spl - v0])
                    stag[pl.ds(pl.multiple_of(slot * 64 + j * L, L), L)] = row
                pltpu.async_copy(
                    stag.at[pl.ds(slot * 64, 64)],
                    out_hbm.at[pl.ds(b_s * 128, 64)],
                    wsem,
                )
                return fired + 1

            return lax.fori_loop(0, m_batch, one, fired)

        def window_pairs(v0, width, fired, from_tail):
            def rescan(t, cnt2):
                v = pv[pl.ds(pl.multiple_of(t * L, L), L)]
                b = pb[pl.ds(pl.multiple_of(t * L, L), L)]
                m = (v >= v0) & (v < v0 + width)
                plsc.store_compressed(wv.at[pl.ds(cnt2, L)], v, mask=m)
                plsc.store_compressed(wb.at[pl.ds(cnt2, L)], b, mask=m)
                u = v - v0
                ge = 1 - jnp.minimum(jnp.maximum(-u, 0), 1)
                lt = 1 - jnp.minimum(jnp.maximum(u - (width - 1), 0), 1)
                return cnt2 + jnp.sum(ge * lt)

            m_w = lax.fori_loop(0, n_scan, rescan, 0)

            return pair_batch(m_w, v0, fired, from_tail)

        n_win = _STRIPE_BLKS + jnp.where(wid == NW - 1, 4, 0)

        def window(g, fired):
            blk = wid * _STRIPE_BLKS + g
            v0 = blk * _BLK
            pltpu.sync_copy(wt_hbm.at[:, pl.ds(v0, _BLK)], chunk)
            return window_pairs(v0, _BLK, fired, False)

        fired = lax.fori_loop(0, n_win, window, 0)

        fired = lax.cond(
            wid == NW - 1,
            lambda f: window_pairs(tail0, V - tail0, f, True),
            lambda f: f,
            fired,
        )

        def drain(i, _):
            pltpu.make_async_copy(
                out_hbm.at[pl.ds(0, 64)],
                stag.at[pl.ds(0, 64)],
                wsem,
            ).wait()
            return _

        lax.fori_loop(0, jnp.minimum(fired, _RING), drain, None)

    return k


def kernel(global_state, embed_weight):
    B, = global_state.shape
    V, D = embed_weight.shape
    wt = embed_weight.T
    tail0 = (V // 128) * 128
    tail = embed_weight[tail0:, :]
    out1 = _build(B, V, D)(global_state.astype(jnp.int32), wt, tail)
    return out1.reshape(B, 128)[:, :D]


# speedup vs baseline: 23.8257x; 2.1539x over previous
import functools
import jax
import jax.numpy as jnp
from jax import lax
from jax.experimental import pallas as pl
from jax.experimental.pallas import tpu as pltpu
from jax.experimental.pallas import tpu_sc as plsc

_BLK = 256
_STRIPE_BLKS = 122
_RING = 64


def _build(B, V, D):
    info = plsc.get_sparse_core_info()
    NC, NS, L = info.num_cores, info.num_subcores, info.num_lanes
    NW = NC * NS
    stripe_v = _STRIPE_BLKS * _BLK
    tail0 = (V // _BLK) * _BLK
    cap = B + _RING

    mesh = plsc.VectorSubcoreMesh(core_axis_name="c", subcore_axis_name="s")

    @functools.partial(
        pl.kernel,
        mesh=mesh,
        out_type=jax.ShapeDtypeStruct((B * 128,), jnp.float32),
        scratch_types=[
            pltpu.VMEM((B,), jnp.int32),
            pltpu.VMEM((cap,), jnp.int32),
            pltpu.VMEM((cap,), jnp.int32),
            pltpu.VMEM((cap,), jnp.int32),
            pltpu.VMEM((cap,), jnp.int32),
            pltpu.VMEM((2, 64, _BLK), jnp.float32),
            pltpu.VMEM((V - tail0, 64), jnp.float32),
            pltpu.VMEM((_RING * 64,), jnp.float32),
            pltpu.SMEM((64,), jnp.int32),
            pltpu.SMEM((64,), jnp.int32),
            pltpu.SemaphoreType.DMA,
            pltpu.SemaphoreType.DMA,
            pltpu.SemaphoreType.DMA,
        ],
        compiler_params=pltpu.CompilerParams(needs_layout_passes=False),
    )
    def k(idx_hbm, wt_hbm, tail_hbm, out_hbm,
          idx_v, pv, pb, wv, wb, chunk, tailb, stag, sv, sb,
          lsemA, lsemB, wsem):
        wid = lax.axis_index("s") * NC + lax.axis_index("c")
        pltpu.sync_copy(idx_hbm, idx_v)
        pltpu.sync_copy(tail_hbm, tailb)

        iota = lax.iota(jnp.int32, L)

        def prescan(j, cnt):
            v = idx_v[pl.ds(pl.multiple_of(j * L, L), L)]
            q = jnp.minimum(v // stripe_v, NW - 1)
            m = q == wid
            plsc.store_compressed(pv.at[pl.ds(cnt, L)], v, mask=m)
            plsc.store_compressed(pb.at[pl.ds(cnt, L)], j * L + iota, mask=m)
            return cnt + jnp.sum(1 - jnp.minimum(jnp.abs(q - wid), 1))

        n_w = lax.fori_loop(0, B // L, prescan, 0)
        n_scan = pl.cdiv(n_w, L)

        def pair_batch(m_batch, v0, fired, from_tail, buf):
            def one(i, fired):
                al = pl.multiple_of((i >> 4) * L, L)
                lane = jnp.full((L,), i & (L - 1), jnp.int32)
                v_spl = jnp.take(wv[pl.ds(al, L)], lane)
                b_s = jnp.take(wb[pl.ds(al, L)], lane)[0]
                slot = lax.rem(fired, _RING)

                @pl.when(fired >= _RING)
                def _():
                    pltpu.make_async_copy(
                        out_hbm.at[pl.ds(0, 64)],
                        stag.at[pl.ds(0, 64)],
                        wsem,
                    ).wait()

                for j in range(4):
                    if from_tail:
                        row = plsc.load_gather(
                            tailb, [v_spl - v0, iota + j * L])
                    else:
                        row = plsc.load_gather(
                            buf, [iota + j * L, v_spl - v0])
                    stag[pl.ds(pl.multiple_of(slot * 64 + j * L, L), L)] = row
                pltpu.async_copy(
                    stag.at[pl.ds(slot * 64, 64)],
                    out_hbm.at[pl.ds(b_s * 128, 64)],
                    wsem,
                )
                return fired + 1

            return lax.fori_loop(0, m_batch, one, fired)

        def window_pairs(v0, width, fired, from_tail, buf):
            def rescan(t, cnt2):
                v = pv[pl.ds(pl.multiple_of(t * L, L), L)]
                b = pb[pl.ds(pl.multiple_of(t * L, L), L)]
                m = (v >= v0) & (v < v0 + width)
                plsc.store_compressed(wv.at[pl.ds(cnt2, L)], v, mask=m)
                plsc.store_compressed(wb.at[pl.ds(cnt2, L)], b, mask=m)
                u = v - v0
                ge = 1 - jnp.minimum(jnp.maximum(-u, 0), 1)
                lt = 1 - jnp.minimum(jnp.maximum(u - (width - 1), 0), 1)
                return cnt2 + jnp.sum(ge * lt)

            m_w = lax.fori_loop(0, n_scan, rescan, 0)

            return pair_batch(m_w, v0, fired, from_tail, buf)

        n_win = _STRIPE_BLKS + jnp.where(wid == NW - 1, 2, 0)
        stripe0 = wid * _STRIPE_BLKS * _BLK

        def start(g, sem):
            pltpu.async_copy(
                wt_hbm.at[:, pl.ds(stripe0 + g * _BLK, _BLK)],
                chunk.at[lax.rem(g, 2)],
                sem,
            )

        def wait_win(sem):
            pltpu.make_async_copy(
                wt_hbm.at[:, pl.ds(0, _BLK)], chunk.at[0], sem,
            ).wait()

        start(0, lsemA)

        def gpair(g2, fired):
            g = g2 * 2

            @pl.when(g + 1 < n_win)
            def _():
                start(g + 1, lsemB)

            wait_win(lsemA)
            fired = window_pairs(stripe0 + g * _BLK, _BLK, fired, False,
                                 chunk.at[0])

            @pl.when(g + 2 < n_win)
            def _():
                start(g + 2, lsemA)

            wait_win(lsemB)
            fired = window_pairs(stripe0 + (g + 1) * _BLK, _BLK, fired,
                                 False, chunk.at[1])
            return fired

        fired = lax.fori_loop(0, n_win // 2, gpair, 0)

        fired = lax.cond(
            wid == NW - 1,
            lambda f: window_pairs(tail0, V - tail0, f, True, chunk.at[0]),
            lambda f: f,
            fired,
        )

        def drain(i, _):
            pltpu.make_async_copy(
                out_hbm.at[pl.ds(0, 64)],
                stag.at[pl.ds(0, 64)],
                wsem,
            ).wait()
            return _

        lax.fori_loop(0, jnp.minimum(fired, _RING), drain, None)

    return k


def kernel(global_state, embed_weight):
    B, = global_state.shape
    V, D = embed_weight.shape
    wt = embed_weight.T
    tail0 = (V // 128) * 128
    tail = embed_weight[tail0:, :]
    out1 = _build(B, V, D)(global_state.astype(jnp.int32), wt, tail)
    return out1.reshape(B, 128)[:, :D]


# speedup vs baseline: 26.2837x; 1.1032x over previous
import functools
import jax
import jax.numpy as jnp
from jax import lax
from jax.experimental import pallas as pl
from jax.experimental.pallas import tpu as pltpu
from jax.experimental.pallas import tpu_sc as plsc

_BLK = 512
_NWIN = 61
_RING = 32
_SEG = 2048


def _build(B, V, D):
    info = plsc.get_sparse_core_info()
    NC, NS, L = info.num_cores, info.num_subcores, info.num_lanes
    NW = NC * NS
    stripe_v = _NWIN * _BLK
    tail0 = (V // 128) * 128
    cap = B + _RING

    mesh = plsc.VectorSubcoreMesh(core_axis_name="c", subcore_axis_name="s")

    @functools.partial(
        pl.kernel,
        mesh=mesh,
        out_type=jax.ShapeDtypeStruct((B * 128,), jnp.float32),
        scratch_types=[
            pltpu.VMEM((B,), jnp.int32),
            pltpu.VMEM((cap,), jnp.int32),
            pltpu.VMEM((cap,), jnp.int32),
            pltpu.VMEM((_SEG + L,), jnp.int32),
            pltpu.VMEM((_SEG + L,), jnp.int32),
            pltpu.VMEM((2, 64, _BLK), jnp.float32),
            pltpu.VMEM((V - tail0, 64), jnp.float32),
            pltpu.VMEM((_RING * 64,), jnp.float32),
            pltpu.SemaphoreType.DMA,
            pltpu.SemaphoreType.DMA,
            pltpu.SemaphoreType.DMA,
        ],
        compiler_params=pltpu.CompilerParams(needs_layout_passes=False),
    )
    def k(idx_hbm, wt_hbm, tail_hbm, out_hbm,
          idx_v, pv, pb, wv, wb, chunk, tailb, stag,
          lsemA, lsemB, wsem):
        wid = lax.axis_index("s") * NC + lax.axis_index("c")
        pltpu.sync_copy(idx_hbm, idx_v)
        pltpu.sync_copy(tail_hbm, tailb)
        iota = lax.iota(jnp.int32, L)

        def prescan(j, cnt):
            v = idx_v[pl.ds(pl.multiple_of(j * L, L), L)]
            q = jnp.minimum(v // stripe_v, NW - 1)
            m = q == wid
            plsc.store_compressed(pv.at[pl.ds(cnt, L)], v, mask=m)
            plsc.store_compressed(pb.at[pl.ds(cnt, L)], j * L + iota, mask=m)
            return cnt + jnp.sum(1 - jnp.minimum(jnp.abs(q - wid), 1))

        n_w = lax.fori_loop(0, B // L, prescan, 0)

        def pair_loop(m_b, v0, fired, from_tail, buf):
            def one(i, fired):
                al = pl.multiple_of((i >> 4) * L, L)
                lane = jnp.full((L,), i & (L - 1), jnp.int32)
                v_spl = jnp.take(wv[pl.ds(al, L)], lane)
                b_s = jnp.take(wb[pl.ds(al, L)], lane)[0]
                slot = lax.rem(fired, _RING)

                @pl.when(fired >= _RING)
                def _():
                    pltpu.make_async_copy(
                        out_hbm.at[pl.ds(0, 64)],
                        stag.at[pl.ds(0, 64)],
                        wsem,
                    ).wait()

                for j in range(4):
                    if from_tail:
                        row = plsc.load_gather(
                            tailb, [v_spl - v0, iota + j * L])
                    else:
                        row = plsc.load_gather(
                            buf, [iota + j * L, v_spl - v0])
                    stag[pl.ds(pl.multiple_of(slot * 64 + j * L, L), L)] = row
                pltpu.async_copy(
                    stag.at[pl.ds(slot * 64, 64)],
                    out_hbm.at[pl.ds(b_s * 128, 64)],
                    wsem,
                )
                return fired + 1

            return lax.fori_loop(0, m_b, one, fired)

        def window_pairs(v0, width, fired, from_tail, buf):
            def seg(sg, fired):
                p0 = sg * _SEG
                n_in = jnp.minimum(n_w - p0, _SEG)

                def refilter(t, cnt2):
                    off = pl.multiple_of(p0 + t * L, L)
                    pos = off + iota
                    v = pv[pl.ds(off, L)]
                    b = pb[pl.ds(off, L)]
                    m = (v >= v0) & (v < v0 + width) & (pos < n_w)
                    plsc.store_compressed(wv.at[pl.ds(cnt2, L)], v, mask=m)
                    plsc.store_compressed(wb.at[pl.ds(cnt2, L)], b, mask=m)
                    u = v - v0
                    ge = 1 - jnp.minimum(jnp.maximum(-u, 0), 1)
                    lt = 1 - jnp.minimum(jnp.maximum(u - (width - 1), 0), 1)
                    ok = 1 - jnp.minimum(jnp.maximum(pos - (n_w - 1), 0), 1)
                    return cnt2 + jnp.sum(ge * lt * ok)

                m_b = lax.fori_loop(0, (n_in + L - 1) // L, refilter, 0)
                return pair_loop(m_b, v0, fired, from_tail, buf)

            return lax.fori_loop(0, (n_w + _SEG - 1) // _SEG, seg, fired)

        n_win = _NWIN + jnp.where(wid == NW - 1, 1, 0)
        stripe0 = wid * stripe_v

        def start(g, sem):
            pltpu.async_copy(
                wt_hbm.at[:, pl.ds(stripe0 + g * _BLK, _BLK)],
                chunk.at[lax.rem(g, 2)],
                sem,
            )

        def wait_win(sem):
            pltpu.make_async_copy(
                wt_hbm.at[:, pl.ds(0, _BLK)], chunk.at[0], sem,
            ).wait()

        start(0, lsemA)

        def gpair(g2, fired):
            g = g2 * 2

            @pl.when(g + 1 < n_win)
            def _():
                start(g + 1, lsemB)

            wait_win(lsemA)
            fired = window_pairs(stripe0 + g * _BLK, _BLK, fired, False,
                                 chunk.at[0])

            @pl.when(g + 2 < n_win)
            def _():
                start(g + 2, lsemA)

            @pl.when(g + 1 < n_win)
            def _():
                wait_win(lsemB)

            fired = lax.cond(
                g + 1 < n_win,
                lambda f: window_pairs(stripe0 + (g + 1) * _BLK, _BLK, f,
                                       False, chunk.at[1]),
                lambda f: f,
                fired,
            )
            return fired

        fired = lax.fori_loop(0, (_NWIN + 1) // 2, gpair, 0)

        fired = lax.cond(
            wid == NW - 1,
            lambda f: window_pairs(tail0, V - tail0, f, True, chunk.at[0]),
            lambda f: f,
            fired,
        )

        def drain(i, _):
            pltpu.make_async_copy(
                out_hbm.at[pl.ds(0, 64)],
                stag.at[pl.ds(0, 64)],
                wsem,
            ).wait()
            return _

        lax.fori_loop(0, jnp.minimum(fired, _RING), drain, None)

    return k


def kernel(global_state, embed_weight):
    B, = global_state.shape
    V, D = embed_weight.shape
    wt = embed_weight.T
    tail0 = (V // 128) * 128
    tail = embed_weight[tail0:, :]
    out1 = _build(B, V, D)(global_state.astype(jnp.int32), wt, tail)
    return out1.reshape(B, 128)[:, :D]


# speedup vs baseline: 27.3399x; 1.0402x over previous
import functools
import jax
import jax.numpy as jnp
from jax import lax
from jax.experimental import pallas as pl
from jax.experimental.pallas import tpu as pltpu
from jax.experimental.pallas import tpu_sc as plsc

_BLK = 256
_NWIN = 122
_RING = 32
_SEG = 2048


def _build(B, V, D):
    info = plsc.get_sparse_core_info()
    NC, NS, L = info.num_cores, info.num_subcores, info.num_lanes
    NW = NC * NS
    stripe_v = _NWIN * _BLK
    tail0 = (V // 128) * 128
    cap = B + _RING

    mesh = plsc.VectorSubcoreMesh(core_axis_name="c", subcore_axis_name="s")

    @functools.partial(
        pl.kernel,
        mesh=mesh,
        out_type=jax.ShapeDtypeStruct((B * 128,), jnp.float32),
        scratch_types=[
            pltpu.VMEM((B,), jnp.int32),
            pltpu.VMEM((cap,), jnp.int32),
            pltpu.VMEM((cap,), jnp.int32),
            pltpu.VMEM((_SEG + L,), jnp.int32),
            pltpu.VMEM((_SEG + L,), jnp.int32),
            pltpu.VMEM((4, 64, _BLK), jnp.float32),
            pltpu.VMEM((V - tail0, 64), jnp.float32),
            pltpu.VMEM((_RING * 64,), jnp.float32),
            pltpu.SemaphoreType.DMA,
            pltpu.SemaphoreType.DMA,
            pltpu.SemaphoreType.DMA,
            pltpu.SemaphoreType.DMA,
            pltpu.SemaphoreType.DMA,
        ],
        compiler_params=pltpu.CompilerParams(needs_layout_passes=False),
    )
    def k(idx_hbm, wt_hbm, tail_hbm, out_hbm,
          idx_v, pv, pb, wv, wb, chunk, tailb, stag,
          lsem0, lsem1, lsem2, lsem3, wsem):
        wid = lax.axis_index("s") * NC + lax.axis_index("c")
        pltpu.sync_copy(idx_hbm, idx_v)
        pltpu.sync_copy(tail_hbm, tailb)
        iota = lax.iota(jnp.int32, L)

        def prescan(j, cnt):
            v = idx_v[pl.ds(pl.multiple_of(j * L, L), L)]
            q = jnp.minimum(v // stripe_v, NW - 1)
            m = q == wid
            plsc.store_compressed(pv.at[pl.ds(cnt, L)], v, mask=m)
            plsc.store_compressed(pb.at[pl.ds(cnt, L)], j * L + iota, mask=m)
            return cnt + jnp.sum(1 - jnp.minimum(jnp.abs(q - wid), 1))

        n_w = lax.fori_loop(0, B // L, prescan, 0)

        def pair_loop(m_b, v0, fired, from_tail, buf):
            def one(i, fired):
                al = pl.multiple_of((i >> 4) * L, L)
                lane = jnp.full((L,), i & (L - 1), jnp.int32)
                v_spl = jnp.take(wv[pl.ds(al, L)], lane)
                b_s = jnp.take(wb[pl.ds(al, L)], lane)[0]
                slot = lax.rem(fired, _RING)

                @pl.when(fired >= _RING)
                def _():
                    pltpu.make_async_copy(
                        out_hbm.at[pl.ds(0, 64)],
                        stag.at[pl.ds(0, 64)],
                        wsem,
                    ).wait()

                for j in range(4):
                    if from_tail:
                        row = plsc.load_gather(
                            tailb, [v_spl - v0, iota + j * L])
                    else:
                        row = plsc.load_gather(
                            buf, [iota + j * L, v_spl - v0])
                    stag[pl.ds(pl.multiple_of(slot * 64 + j * L, L), L)] = row
                pltpu.async_copy(
                    stag.at[pl.ds(slot * 64, 64)],
                    out_hbm.at[pl.ds(b_s * 128, 64)],
                    wsem,
                )
                return fired + 1

            return lax.fori_loop(0, m_b, one, fired)

        def window_pairs(v0, width, fired, from_tail, buf):
            def seg(sg, fired):
                p0 = sg * _SEG
                n_in = jnp.minimum(n_w - p0, _SEG)

                def refilter(t, cnt2):
                    off = pl.multiple_of(p0 + t * L, L)
                    pos = off + iota
                    v = pv[pl.ds(off, L)]
                    b = pb[pl.ds(off, L)]
                    m = (v >= v0) & (v < v0 + width) & (pos < n_w)
                    plsc.store_compressed(wv.at[pl.ds(cnt2, L)], v, mask=m)
                    plsc.store_compressed(wb.at[pl.ds(cnt2, L)], b, mask=m)
                    u = v - v0
                    ge = 1 - jnp.minimum(jnp.maximum(-u, 0), 1)
                    lt = 1 - jnp.minimum(jnp.maximum(u - (width - 1), 0), 1)
                    ok = 1 - jnp.minimum(jnp.maximum(pos - (n_w - 1), 0), 1)
                    return cnt2 + jnp.sum(ge * lt * ok)

                m_b = lax.fori_loop(0, (n_in + L - 1) // L, refilter, 0)
                return pair_loop(m_b, v0, fired, from_tail, buf)

            return lax.fori_loop(0, (n_w + _SEG - 1) // _SEG, seg, fired)

        n_win = _NWIN + jnp.where(wid == NW - 1, 2, 0)
        stripe0 = wid * stripe_v
        sems = [lsem0, lsem1, lsem2, lsem3]

        def start(g, k):
            pltpu.async_copy(
                wt_hbm.at[:, pl.ds(stripe0 + g * _BLK, _BLK)],
                chunk.at[k],
                sems[k],
            )

        def wait_win(k):
            pltpu.make_async_copy(
                wt_hbm.at[:, pl.ds(0, _BLK)], chunk.at[0], sems[k],
            ).wait()

        for k in range(4):
            start(k, k)

        def quad(q, fired):
            g0 = q * 4
            for k in range(4):
                g = g0 + k

                def do(f, g=g, k=k):
                    wait_win(k)
                    f = window_pairs(stripe0 + g * _BLK, _BLK, f, False,
                                     chunk.at[k])

                    @pl.when(g + 4 < n_win)
                    def _():
                        start(g + 4, k)

                    return f

                fired = lax.cond(g < n_win, do, lambda f: f, fired)
            return fired

        fired = lax.fori_loop(0, (_NWIN + 2 + 3) // 4, quad, 0)

        fired = lax.cond(
            wid == NW - 1,
            lambda f: window_pairs(tail0, V - tail0, f, True, chunk.at[0]),
            lambda f: f,
            fired,
        )

        def drain(i, _):
            pltpu.make_async_copy(
                out_hbm.at[pl.ds(0, 64)],
                stag.at[pl.ds(0, 64)],
                wsem,
            ).wait()
            return _

        lax.fori_loop(0, jnp.minimum(fired, _RING), drain, None)

    return k


def kernel(global_state, embed_weight):
    B, = global_state.shape
    V, D = embed_weight.shape
    wt = embed_weight.T
    tail0 = (V // 128) * 128
    tail = embed_weight[tail0:, :]
    out1 = _build(B, V, D)(global_state.astype(jnp.int32), wt, tail)
    return out1.reshape(B, 128)[:, :D]


# speedup vs baseline: 27.8159x; 1.0174x over previous
import functools
import jax
import jax.numpy as jnp
from jax import lax
from jax.experimental import pallas as pl
from jax.experimental.pallas import tpu as pltpu
from jax.experimental.pallas import tpu_sc as plsc

_BLK = 256
_NWIN = 122
_RING = 32
_SEG = 2048


def _build(B, V, D):
    info = plsc.get_sparse_core_info()
    NC, NS, L = info.num_cores, info.num_subcores, info.num_lanes
    NW = NC * NS
    stripe_v = _NWIN * _BLK
    tail0 = (V // 128) * 128
    cap = B + _RING

    mesh = plsc.VectorSubcoreMesh(core_axis_name="c", subcore_axis_name="s")

    @functools.partial(
        pl.kernel,
        mesh=mesh,
        out_type=jax.ShapeDtypeStruct((B * 128,), jnp.float32),
        scratch_types=[
            pltpu.VMEM((B,), jnp.int32),
            pltpu.VMEM((cap,), jnp.int32),
            pltpu.VMEM((cap,), jnp.int32),
            pltpu.VMEM((_SEG + L,), jnp.int32),
            pltpu.VMEM((_SEG + L,), jnp.int32),
            pltpu.VMEM((4, 64, _BLK), jnp.float32),
            pltpu.VMEM((V - tail0, 64), jnp.float32),
            pltpu.VMEM((_RING * 64,), jnp.float32),
            pltpu.SemaphoreType.DMA,
            pltpu.SemaphoreType.DMA,
            pltpu.SemaphoreType.DMA,
            pltpu.SemaphoreType.DMA,
            pltpu.SemaphoreType.DMA,
        ],
        compiler_params=pltpu.CompilerParams(needs_layout_passes=False),
    )
    def k(idx_hbm, wt_hbm, tail_hbm, out_hbm,
          idx_v, pv, pb, wv, wb, chunk, tailb, stag,
          lsem0, lsem1, lsem2, lsem3, wsem):
        wid = lax.axis_index("s") * NC + lax.axis_index("c")
        pltpu.sync_copy(idx_hbm, idx_v)
        pltpu.sync_copy(tail_hbm, tailb)
        iota = lax.iota(jnp.int32, L)

        n_win = _NWIN + jnp.where(wid == NW - 1, 2, 0)
        stripe0 = wid * stripe_v
        sems = [lsem0, lsem1, lsem2, lsem3]

        def start(g, k):
            pltpu.async_copy(
                wt_hbm.at[:, pl.ds(stripe0 + g * _BLK, _BLK)],
                chunk.at[k],
                sems[k],
            )

        def wait_win(k):
            pltpu.make_async_copy(
                wt_hbm.at[:, pl.ds(0, _BLK)], chunk.at[0], sems[k],
            ).wait()

        for k in range(4):
            start(k, k)

        def prescan(j, cnt):
            v = idx_v[pl.ds(pl.multiple_of(j * L, L), L)]
            q = jnp.minimum(v // stripe_v, NW - 1)
            m = q == wid
            plsc.store_compressed(pv.at[pl.ds(cnt, L)], v, mask=m)
            plsc.store_compressed(pb.at[pl.ds(cnt, L)], j * L + iota, mask=m)
            return cnt + jnp.sum(1 - jnp.minimum(jnp.abs(q - wid), 1))

        n_w = lax.fori_loop(0, B // L, prescan, 0)

        def pair_loop(m_b, v0, fired, from_tail, buf):
            def one(i, fired):
                al = pl.multiple_of((i >> 4) * L, L)
                lane = jnp.full((L,), i & (L - 1), jnp.int32)
                v_spl = jnp.take(wv[pl.ds(al, L)], lane)
                b_s = jnp.take(wb[pl.ds(al, L)], lane)[0]
                slot = lax.rem(fired, _RING)

                @pl.when(fired >= _RING)
                def _():
                    pltpu.make_async_copy(
                        out_hbm.at[pl.ds(0, 64)],
                        stag.at[pl.ds(0, 64)],
                        wsem,
                    ).wait()

                for j in range(4):
                    if from_tail:
                        row = plsc.load_gather(
                            tailb, [v_spl - v0, iota + j * L])
                    else:
                        row = plsc.load_gather(
                            buf, [iota + j * L, v_spl - v0])
                    stag[pl.ds(pl.multiple_of(slot * 64 + j * L, L), L)] = row
                pltpu.async_copy(
                    stag.at[pl.ds(slot * 64, 64)],
                    out_hbm.at[pl.ds(b_s * 128, 64)],
                    wsem,
                )
                return fired + 1

            return lax.fori_loop(0, m_b, one, fired)

        def window_pairs(v0, width, fired, from_tail, buf):
            def seg(sg, fired):
                p0 = sg * _SEG
                n_in = jnp.minimum(n_w - p0, _SEG)

                def refilter(t, cnt2):
                    off = pl.multiple_of(p0 + t * L, L)
                    pos = off + iota
                    v = pv[pl.ds(off, L)]
                    b = pb[pl.ds(off, L)]
                    m = (v >= v0) & (v < v0 + width) & (pos < n_w)
                    plsc.store_compressed(wv.at[pl.ds(cnt2, L)], v, mask=m)
                    plsc.store_compressed(wb.at[pl.ds(cnt2, L)], b, mask=m)
                    u = v - v0
                    ge = 1 - jnp.minimum(jnp.maximum(-u, 0), 1)
                    lt = 1 - jnp.minimum(jnp.maximum(u - (width - 1), 0), 1)
                    ok = 1 - jnp.minimum(jnp.maximum(pos - (n_w - 1), 0), 1)
                    return cnt2 + jnp.sum(ge * lt * ok)

                m_b = lax.fori_loop(0, (n_in + L - 1) // L, refilter, 0)
                return pair_loop(m_b, v0, fired, from_tail, buf)

            return lax.fori_loop(0, (n_w + _SEG - 1) // _SEG, seg, fired)

        def quad(q, fired):
            g0 = q * 4
            for k in range(4):
                g = g0 + k

                def do(f, g=g, k=k):
                    wait_win(k)
                    f = window_pairs(stripe0 + g * _BLK, _BLK, f, False,
                                     chunk.at[k])

                    @pl.when(g + 4 < n_win)
                    def _():
                        start(g + 4, k)

                    return f

                fired = lax.cond(g < n_win, do, lambda f: f, fired)
            return fired

        fired = lax.fori_loop(0, (_NWIN + 2 + 3) // 4, quad, 0)

        fired = lax.cond(
            wid == NW - 1,
            lambda f: window_pairs(tail0, V - tail0, f, True, chunk.at[0]),
            lambda f: f,
            fired,
        )

        def drain(i, _):
            pltpu.make_async_copy(
                out_hbm.at[pl.ds(0, 64)],
                stag.at[pl.ds(0, 64)],
                wsem,
            ).wait()
            return _

        lax.fori_loop(0, jnp.minimum(fired, _RING), drain, None)

    return k


def kernel(global_state, embed_weight):
    B, = global_state.shape
    V, D = embed_weight.shape
    wt = embed_weight.T
    tail0 = (V // 128) * 128
    tail = embed_weight[tail0:, :]
    out1 = _build(B, V, D)(global_state.astype(jnp.int32), wt, tail)
    return out1.reshape(B, 128)[:, :D]


# speedup vs baseline: 35.3410x; 1.2705x over previous
import functools
import jax
import jax.numpy as jnp
from jax import lax
from jax.experimental import pallas as pl
from jax.experimental.pallas import tpu as pltpu
from jax.experimental.pallas import tpu_sc as plsc

_BLK = 256
_NWIN = 122
_RING = 32
_SEG = 2048


def _build(B, V, D):
    info = plsc.get_sparse_core_info()
    NC, NS, L = info.num_cores, info.num_subcores, info.num_lanes
    NW = NC * NS
    stripe_v = _NWIN * _BLK
    tail0 = (V // 128) * 128
    cap = B + _RING

    mesh = plsc.VectorSubcoreMesh(core_axis_name="c", subcore_axis_name="s")

    @functools.partial(
        pl.kernel,
        mesh=mesh,
        out_type=jax.ShapeDtypeStruct((B * 128,), jnp.float32),
        scratch_types=[
            pltpu.VMEM((B,), jnp.int32),
            pltpu.VMEM((cap,), jnp.int32),
            pltpu.VMEM((cap,), jnp.int32),
            pltpu.VMEM((_SEG + L,), jnp.int32),
            pltpu.VMEM((_SEG + L,), jnp.int32),
            pltpu.VMEM((4, 64, _BLK), jnp.float32),
            pltpu.VMEM((V - tail0, 64), jnp.float32),
            pltpu.VMEM((_RING * 64,), jnp.float32),
            pltpu.SemaphoreType.DMA,
            pltpu.SemaphoreType.DMA,
            pltpu.SemaphoreType.DMA,
            pltpu.SemaphoreType.DMA,
            pltpu.SemaphoreType.DMA,
        ],
        compiler_params=pltpu.CompilerParams(needs_layout_passes=False),
    )
    def k(idx_hbm, wt_hbm, tail_hbm, out_hbm,
          idx_v, pv, pb, wv, wb, chunk, tailb, stag,
          lsem0, lsem1, lsem2, lsem3, wsem):
        wid = lax.axis_index("s") * NC + lax.axis_index("c")
        iota = lax.iota(jnp.int32, L)

        n_win = _NWIN + jnp.where(wid == NW - 1, 2, 0)
        stripe0 = wid * stripe_v
        sems = [lsem0, lsem1, lsem2, lsem3]

        def start(g, k):
            pltpu.async_copy(
                wt_hbm.at[:, pl.ds(stripe0 + g * _BLK, _BLK)],
                chunk.at[k],
                sems[k],
            )

        def wait_win(k):
            pltpu.make_async_copy(
                wt_hbm.at[:, pl.ds(0, _BLK)], chunk.at[0], sems[k],
            ).wait()

        for k in range(4):
            start(k, k)
        pltpu.sync_copy(idx_hbm, idx_v)
        pltpu.sync_copy(tail_hbm, tailb)

        def prescan(j, cnt):
            v = idx_v[pl.ds(pl.multiple_of(j * L, L), L)]
            q = jnp.minimum(((v >> 9) * 68760) >> 22, NW - 1)
            m = q == wid
            plsc.store_compressed(pv.at[pl.ds(cnt, L)], v, mask=m)
            plsc.store_compressed(pb.at[pl.ds(cnt, L)], j * L + iota, mask=m)
            return cnt + jnp.sum(1 - jnp.minimum(jnp.abs(q - wid), 1))

        n_w = lax.fori_loop(0, B // L, prescan, 0)

        def pair_loop(m_b, v0, fired, from_tail, buf):
            def one(i, fired):
                al = pl.multiple_of((i >> 4) * L, L)
                lane = jnp.full((L,), i & (L - 1), jnp.int32)
                v_spl = jnp.take(wv[pl.ds(al, L)], lane)
                b_s = jnp.take(wb[pl.ds(al, L)], lane)[0]
                slot = lax.rem(fired, _RING)

                @pl.when(fired >= _RING)
                def _():
                    pltpu.make_async_copy(
                        out_hbm.at[pl.ds(0, 64)],
                        stag.at[pl.ds(0, 64)],
                        wsem,
                    ).wait()

                for j in range(4):
                    if from_tail:
                        row = plsc.load_gather(
                            tailb, [v_spl - v0, iota + j * L])
                    else:
                        row = plsc.load_gather(
                            buf, [iota + j * L, v_spl - v0])
                    stag[pl.ds(pl.multiple_of(slot * 64 + j * L, L), L)] = row
                pltpu.async_copy(
                    stag.at[pl.ds(slot * 64, 64)],
                    out_hbm.at[pl.ds(b_s * 128, 64)],
                    wsem,
                )
                return fired + 1

            return lax.fori_loop(0, m_b, one, fired)

        def window_pairs(v0, width, fired, from_tail, buf):
            def seg(sg, fired):
                p0 = sg * _SEG
                n_in = jnp.minimum(n_w - p0, _SEG)

                def refilter(t, cnt2):
                    off = pl.multiple_of(p0 + t * L, L)
                    pos = off + iota
                    v = pv[pl.ds(off, L)]
                    b = pb[pl.ds(off, L)]
                    m = (v >= v0) & (v < v0 + width) & (pos < n_w)
                    plsc.store_compressed(wv.at[pl.ds(cnt2, L)], v, mask=m)
                    plsc.store_compressed(wb.at[pl.ds(cnt2, L)], b, mask=m)
                    u = v - v0
                    ge = 1 - jnp.minimum(jnp.maximum(-u, 0), 1)
                    lt = 1 - jnp.minimum(jnp.maximum(u - (width - 1), 0), 1)
                    ok = 1 - jnp.minimum(jnp.maximum(pos - (n_w - 1), 0), 1)
                    return cnt2 + jnp.sum(ge * lt * ok)

                m_b = lax.fori_loop(0, (n_in + L - 1) // L, refilter, 0)
                return pair_loop(m_b, v0, fired, from_tail, buf)

            return lax.fori_loop(0, (n_w + _SEG - 1) // _SEG, seg, fired)

        def quad(q, fired):
            g0 = q * 4
            for k in range(4):
                g = g0 + k

                def do(f, g=g, k=k):
                    wait_win(k)
                    f = window_pairs(stripe0 + g * _BLK, _BLK, f, False,
                                     chunk.at[k])

                    @pl.when(g + 4 < n_win)
                    def _():
                        start(g + 4, k)

                    return f

                fired = lax.cond(g < n_win, do, lambda f: f, fired)
            return fired

        fired = lax.fori_loop(0, (_NWIN + 2 + 3) // 4, quad, 0)

        fired = lax.cond(
            wid == NW - 1,
            lambda f: window_pairs(tail0, V - tail0, f, True, chunk.at[0]),
            lambda f: f,
            fired,
        )

        def drain(i, _):
            pltpu.make_async_copy(
                out_hbm.at[pl.ds(0, 64)],
                stag.at[pl.ds(0, 64)],
                wsem,
            ).wait()
            return _

        lax.fori_loop(0, jnp.minimum(fired, _RING), drain, None)

    return k


def kernel(global_state, embed_weight):
    B, = global_state.shape
    V, D = embed_weight.shape
    wt = embed_weight.T
    tail0 = (V // 128) * 128
    tail = embed_weight[tail0:, :]
    out1 = _build(B, V, D)(global_state.astype(jnp.int32), wt, tail)
    return out1.reshape(B, 128)[:, :D]


# speedup vs baseline: 38.7558x; 1.0966x over previous
import functools
import jax
import jax.numpy as jnp
from jax import lax
from jax.experimental import pallas as pl
from jax.experimental.pallas import tpu as pltpu
from jax.experimental.pallas import tpu_sc as plsc

_BLK = 256
_NWIN = 122
_RING = 32
_SEG = 2048


def _build(B, V, D):
    info = plsc.get_sparse_core_info()
    NC, NS, L = info.num_cores, info.num_subcores, info.num_lanes
    NW = NC * NS
    stripe_v = _NWIN * _BLK
    tail0 = (V // 128) * 128
    cap = B + _RING

    mesh = plsc.VectorSubcoreMesh(core_axis_name="c", subcore_axis_name="s")

    @functools.partial(
        pl.kernel,
        mesh=mesh,
        out_type=jax.ShapeDtypeStruct((B * 128,), jnp.float32),
        scratch_types=[
            pltpu.VMEM((B,), jnp.int32),
            pltpu.VMEM((cap,), jnp.int32),
            pltpu.VMEM((cap,), jnp.int32),
            pltpu.VMEM((_SEG + L,), jnp.int32),
            pltpu.VMEM((_SEG + L,), jnp.int32),
            pltpu.VMEM((4, 64, _BLK), jnp.float32),
            pltpu.VMEM((V - tail0, 64), jnp.float32),
            pltpu.VMEM((_RING * 64,), jnp.float32),
            pltpu.SemaphoreType.DMA,
            pltpu.SemaphoreType.DMA,
            pltpu.SemaphoreType.DMA,
            pltpu.SemaphoreType.DMA,
            pltpu.SemaphoreType.DMA,
        ],
        compiler_params=pltpu.CompilerParams(needs_layout_passes=False),
    )
    def k(idx_hbm, wt_hbm, tail_hbm, out_hbm,
          idx_v, pv, pb, wv, wb, chunk, tailb, stag,
          lsem0, lsem1, lsem2, lsem3, wsem):
        wid = lax.axis_index("s") * NC + lax.axis_index("c")
        iota = lax.iota(jnp.int32, L)

        n_win = _NWIN + jnp.where(wid == NW - 1, 2, 0)
        stripe0 = wid * stripe_v
        sems = [lsem0, lsem1, lsem2, lsem3]

        def start(g, k):
            pltpu.async_copy(
                wt_hbm.at[:, pl.ds(stripe0 + g * _BLK, _BLK)],
                chunk.at[k],
                sems[k],
            )

        def wait_win(k):
            pltpu.make_async_copy(
                wt_hbm.at[:, pl.ds(0, _BLK)], chunk.at[0], sems[k],
            ).wait()

        for k in range(4):
            start(k, k)
        pltpu.sync_copy(idx_hbm, idx_v)
        pltpu.sync_copy(tail_hbm, tailb)

        def prescan(j, cnt):
            v = idx_v[pl.ds(pl.multiple_of(j * L, L), L)]
            q = jnp.minimum(((v >> 9) * 68760) >> 22, NW - 1)
            m = q == wid
            plsc.store_compressed(pv.at[pl.ds(cnt, L)], v, mask=m)
            plsc.store_compressed(pb.at[pl.ds(cnt, L)], j * L + iota, mask=m)
            return cnt + plsc.all_reduce_population_count(m)[0]

        n_w = lax.fori_loop(0, B // L, prescan, 0)

        def pair_loop(m_b, v0, fired, from_tail, buf):
            def one(i, fired):
                al = pl.multiple_of((i >> 4) * L, L)
                lane = jnp.full((L,), i & (L - 1), jnp.int32)
                v_spl = jnp.take(wv[pl.ds(al, L)], lane)
                b_s = jnp.take(wb[pl.ds(al, L)], lane)[0]
                slot = lax.rem(fired, _RING)

                @pl.when(fired >= _RING)
                def _():
                    pltpu.make_async_copy(
                        out_hbm.at[pl.ds(0, 64)],
                        stag.at[pl.ds(0, 64)],
                        wsem,
                    ).wait()

                for j in range(4):
                    if from_tail:
                        row = plsc.load_gather(
                            tailb, [v_spl - v0, iota + j * L])
                    else:
                        row = plsc.load_gather(
                            buf, [iota + j * L, v_spl - v0])
                    stag[pl.ds(pl.multiple_of(slot * 64 + j * L, L), L)] = row
                pltpu.async_copy(
                    stag.at[pl.ds(slot * 64, 64)],
                    out_hbm.at[pl.ds(b_s * 128, 64)],
                    wsem,
                )
                return fired + 1

            return lax.fori_loop(0, m_b, one, fired)

        def window_pairs(v0, width, fired, from_tail, buf):
            def seg(sg, fired):
                p0 = sg * _SEG
                n_in = jnp.minimum(n_w - p0, _SEG)

                def refilter(t, cnt2):
                    off = pl.multiple_of(p0 + t * L, L)
                    pos = off + iota
                    v = pv[pl.ds(off, L)]
                    b = pb[pl.ds(off, L)]
                    m = (v >= v0) & (v < v0 + width) & (pos < n_w)
                    plsc.store_compressed(wv.at[pl.ds(cnt2, L)], v, mask=m)
                    plsc.store_compressed(wb.at[pl.ds(cnt2, L)], b, mask=m)
                    return cnt2 + plsc.all_reduce_population_count(m)[0]

                m_b = lax.fori_loop(0, (n_in + L - 1) // L, refilter, 0)
                return pair_loop(m_b, v0, fired, from_tail, buf)

            return lax.fori_loop(0, (n_w + _SEG - 1) // _SEG, seg, fired)

        def quad(q, fired):
            g0 = q * 4
            for k in range(4):
                g = g0 + k

                def do(f, g=g, k=k):
                    wait_win(k)
                    f = window_pairs(stripe0 + g * _BLK, _BLK, f, False,
                                     chunk.at[k])

                    @pl.when(g + 4 < n_win)
                    def _():
                        start(g + 4, k)

                    return f

                fired = lax.cond(g < n_win, do, lambda f: f, fired)
            return fired

        fired = lax.fori_loop(0, (_NWIN + 2 + 3) // 4, quad, 0)

        fired = lax.cond(
            wid == NW - 1,
            lambda f: window_pairs(tail0, V - tail0, f, True, chunk.at[0]),
            lambda f: f,
            fired,
        )

        def drain(i, _):
            pltpu.make_async_copy(
                out_hbm.at[pl.ds(0, 64)],
                stag.at[pl.ds(0, 64)],
                wsem,
            ).wait()
            return _

        lax.fori_loop(0, jnp.minimum(fired, _RING), drain, None)

    return k


def kernel(global_state, embed_weight):
    B, = global_state.shape
    V, D = embed_weight.shape
    wt = embed_weight.T
    tail0 = (V // 128) * 128
    tail = embed_weight[tail0:, :]
    out1 = _build(B, V, D)(global_state.astype(jnp.int32), wt, tail)
    return out1.reshape(B, 128)[:, :D]
